# baseline (speedup 1.0000x reference)
import functools

import jax
import jax.numpy as jnp
from jax import lax
from jax.experimental import pallas as pl
from jax.experimental.pallas import tpu as pltpu
from jax.experimental.pallas import tpu_sc as plsc

N = 10000
E = 320000
R = 8
F = 128
L = 16
LB = 32
NSLICE = F // LB
NSEG = N * R
NC = 2
NS = 16
ROWLEN = 128
NBUF = 8
KROWS = 160
NBLK = KROWS // NBUF
E_PAD = NS * ROWLEN * KROWS
ACC_ROWS = NSEG + L
ZROWS_TILE = ACC_ROWS // NS
ZCH = ZROWS_TILE // 3
SEG_TILE = NSEG // NS
CBLK = NBLK // NC


def _make_sc_agg(interpret=False):
    mesh = plsc.VectorSubcoreMesh(core_axis_name="c", subcore_axis_name="s",
                                  num_cores=NC, num_subcores=NS)

    @functools.partial(
        pl.kernel,
        out_type=jax.ShapeDtypeStruct((NC, NSLICE, NSEG, LB), jnp.int16),
        mesh=mesh,
        interpret=interpret,
        compiler_params=pltpu.CompilerParams(use_tc_tiling_on_sc=False),
        scratch_types=[
            pltpu.VMEM_SHARED((ACC_ROWS, LB), jnp.int16),
            pltpu.VMEM((2, NBUF, 2, ROWLEN), jnp.int32),
            pltpu.VMEM((2, NBUF, ROWLEN, LB), jnp.int16),
            pltpu.SemaphoreType.DMA,
            pltpu.SemaphoreType.DMA,
            pltpu.SemaphoreType.DMA,
        ],
    )
    def sc_kernel(xv_hbm, idx2_hbm, zeros_hbm, a_out,
                  acc_sh, idx_v, row_v, gsem, ssem, isem):
        c = lax.axis_index("c")
        s = lax.axis_index("s")

        def zero_my_stripe():
            for kz in range(3):
                pltpu.sync_copy(
                    zeros_hbm, acc_sh.at[pl.ds(s * ZROWS_TILE + kz * ZCH, ZCH)])

        def iwait():
            pltpu.make_async_copy(idx2_hbm.at[0, 0, 0], idx_v.at[0],
                                  isem).wait()

        def gissue(h):
            for b in range(NBUF):
                pltpu.async_copy(xv_hbm.at[idx_v.at[h, b, 0]],
                                 row_v.at[h, b], gsem)

        def gdrain():
            for _ in range(NBUF):
                pltpu.make_async_copy(xv_hbm.at[pl.ds(0, ROWLEN)],
                                      row_v.at[0, 0], gsem).wait()

        def sissue(h):
            for b in range(NBUF):
                pltpu.async_copy(row_v.at[h, b],
                                 acc_sh.at[idx_v.at[h, b, 1]], ssem, add=True)

        def sdrain():
            for _ in range(NBUF):
                pltpu.make_async_copy(row_v.at[0, 0],
                                      acc_sh.at[pl.ds(0, ROWLEN)], ssem).wait()

        nhb = NBLK // NC
        blk_base = c * nhb

        for i in range(NSLICE):

            def istart(blk, h):
                pltpu.async_copy(idx2_hbm.at[i, s, blk_base + blk],
                                 idx_v.at[h], isem)

            zero_my_stripe()
            plsc.subcore_barrier()

            istart(0, 0)
            iwait()
            gissue(0)
            istart(1, 1)

            def pairbody(ip, carry):
                for h in range(2):
                    jg = ip * 2 + h
                    gdrain()
                    sissue(h)

                    @pl.when(jg + 1 < nhb)
                    def _():
                        iwait()

                    @pl.when(jg + 1 < nhb)
                    def _():
                        gissue(1 - h)

                    sdrain()

                    @pl.when(jg + 2 < nhb)
                    def _():
                        istart(jg + 2, h)
                return carry

            lax.fori_loop(0, nhb // 2, pairbody, 0)
            plsc.subcore_barrier()
            pltpu.sync_copy(acc_sh.at[pl.ds(s * SEG_TILE, SEG_TILE)],
                            a_out.at[c, i, pl.ds(s * SEG_TILE, SEG_TILE)])
            plsc.subcore_barrier()

    return sc_kernel


def _make_sc_cnt(interpret=False):
    mesh = plsc.VectorSubcoreMesh(core_axis_name="c", subcore_axis_name="s",
                                  num_cores=NC, num_subcores=NS)

    @functools.partial(
        pl.kernel,
        out_type=jax.ShapeDtypeStruct((NC, NSEG, L), jnp.float32),
        mesh=mesh,
        interpret=interpret,
        compiler_params=pltpu.CompilerParams(use_tc_tiling_on_sc=False),
        scratch_types=[
            pltpu.VMEM_SHARED((ACC_ROWS, L), jnp.float32),
            pltpu.VMEM((2, NBUF, 2, ROWLEN), jnp.int32),
            pltpu.VMEM((ROWLEN, L), jnp.float32),
            pltpu.SemaphoreType.DMA,
            pltpu.SemaphoreType.DMA,
        ],
    )
    def cnt_kernel(idx2_hbm, ones_hbm, zeros_hbm, cnt_out,
                   acc_sh, idx_v, ones_v, ssem, isem):
        c = lax.axis_index("c")
        s = lax.axis_index("s")
        blk0 = c * CBLK

        pltpu.sync_copy(ones_hbm, ones_v)
        for kz in range(3):
            pltpu.sync_copy(
                zeros_hbm, acc_sh.at[pl.ds(s * ZROWS_TILE + kz * ZCH, ZCH)])
        plsc.subcore_barrier()

        def iwait():
            pltpu.make_async_copy(idx2_hbm.at[0, 0, 0], idx_v.at[0],
                                  isem).wait()

        def sdrain():
            for _ in range(NBUF):
                pltpu.make_async_copy(ones_v, acc_sh.at[pl.ds(0, ROWLEN)],
                                      ssem).wait()

        pltpu.async_copy(idx2_hbm.at[0, s, blk0], idx_v.at[0], isem)

        def cpair(ip, carry):
            for h in range(2):
                jg = ip * 2 + h
                iwait()

                @pl.when(jg + 1 < CBLK)
                def _():
                    pltpu.async_copy(idx2_hbm.at[0, s, blk0 + jg + 1],
                                     idx_v.at[1 - h], isem)

                for b in range(NBUF):
                    pltpu.async_copy(ones_v, acc_sh.at[idx_v.at[h, b, 1]],
                                     ssem, add=True)
                sdrain()
            return carry

        lax.fori_loop(0, CBLK // 2, cpair, 0)
        plsc.subcore_barrier()
        pltpu.sync_copy(acc_sh.at[pl.ds(s * SEG_TILE, SEG_TILE)],
                        cnt_out.at[c, pl.ds(s * SEG_TILE, SEG_TILE)])

    return cnt_kernel


def _prep_stats(xin, c0v, c1v, interpret=False):
    def body(x_ref, c0_ref, c1_ref, mx_ref, mc_ref):
        mx_ref[...] = jnp.max(jnp.abs(x_ref[...])).reshape(1, 1)
        mc_ref[...] = jnp.maximum(jnp.max(c0_ref[...]),
                                  jnp.max(c1_ref[...])).reshape(1, 1)

    return pl.pallas_call(
        body,
        out_shape=[jax.ShapeDtypeStruct((1, 1), jnp.float32),
                   jax.ShapeDtypeStruct((1, 1), jnp.float32)],
        interpret=interpret,
    )(xin, c0v, c1v)


def _tc_layer(a, cnt0, cnt1, inv_s, xin, basis, comp, root, bias, act,
              want_max, interpret=False):
    out_dim = root.shape[1]
    w = jnp.einsum('rb,bio->rio', comp, basis)
    w2 = w.reshape(R, NSLICE, LB, out_dim).transpose(1, 0, 2, 3).reshape(
        NSLICE, R * LB, out_dim)
    a3 = a.reshape(NC, NSLICE, N, R * LB)
    bias2 = bias.reshape(1, out_dim)
    nb = 1000
    grid = (N // nb,)

    def body(a_ref, c0_ref, c1_ref, is_ref, x_ref, w2_ref, root_ref, b_ref,
             *o_refs):
        recip = is_ref[...] / jnp.maximum(c0_ref[...] + c1_ref[...], 1.0)
        acc = jnp.dot(x_ref[...], root_ref[...],
                      preferred_element_type=jnp.float32)
        for p in range(NSLICE):
            ap = (a_ref[0, p].astype(jnp.float32)
                  + a_ref[1, p].astype(jnp.float32)) * recip
            acc += jnp.dot(ap, w2_ref[p], preferred_element_type=jnp.float32)
        acc += b_ref[...]
        if act == 'relu':
            acc = jnp.maximum(acc, 0.0)
        else:
            acc = jnp.tanh(acc)
        o_refs[0][...] = acc
        if want_max:
            i = pl.program_id(0)
            bmax = jnp.max(acc).reshape(1, 1)

            @pl.when(i == 0)
            def _():
                o_refs[1][...] = bmax

            @pl.when(i > 0)
            def _():
                o_refs[1][...] = jnp.maximum(o_refs[1][...], bmax)

    out_shapes = [jax.ShapeDtypeStruct((N, out_dim), jnp.float32)]
    out_specs = [pl.BlockSpec((nb, out_dim), lambda i: (i, 0))]
    if want_max:
        out_shapes.append(jax.ShapeDtypeStruct((1, 1), jnp.float32))
        out_specs.append(pl.BlockSpec((1, 1), lambda i: (0, 0)))

    return pl.pallas_call(
        body,
        grid=grid,
        in_specs=[
            pl.BlockSpec((NC, NSLICE, nb, R * LB), lambda i: (0, 0, i, 0)),
            pl.BlockSpec((nb, R * LB), lambda i: (i, 0)),
            pl.BlockSpec((nb, R * LB), lambda i: (i, 0)),
            pl.BlockSpec((1, 1), lambda i: (0, 0)),
            pl.BlockSpec((nb, F), lambda i: (i, 0)),
            pl.BlockSpec((NSLICE, R * LB, out_dim), lambda i: (0, 0, 0)),
            pl.BlockSpec((F, out_dim), lambda i: (0, 0)),
            pl.BlockSpec((1, out_dim), lambda i: (0, 0)),
        ],
        out_specs=out_specs,
        out_shape=out_shapes,
        interpret=interpret,
    )(a3, cnt0, cnt1, inv_s, xin, w2, root, bias2)


def kernel(x, edge_index, edge_type,
           basis0, comp0, root0, bias0,
           basis1, comp1, root1, bias1,
           basis2, comp2, root2, bias2):
    src = edge_index[0].astype(jnp.int32)
    dst = edge_index[1].astype(jnp.int32)
    seg = dst * R + edge_type.astype(jnp.int32)

    pad = E_PAD - E
    seg_spread = jnp.concatenate(
        [seg, jnp.arange(pad, dtype=jnp.int32) % NSEG]).reshape(
            NS, NBLK, NBUF, ROWLEN)
    seg_trash = jnp.concatenate(
        [seg, jnp.full((pad,), NSEG, jnp.int32)]).reshape(
            NS, NBLK, NBUF, ROWLEN)
    src_p = jnp.concatenate([src, jnp.full((pad,), N, jnp.int32)])
    src_rs = src_p.reshape(NS, NBLK, NBUF, ROWLEN)
    gidx = (src_rs[None] * NSLICE
            + jnp.arange(NSLICE, dtype=jnp.int32)[:, None, None, None, None])
    idx2 = jnp.stack(
        [gidx, jnp.broadcast_to(seg_spread[None], gidx.shape)], axis=4)
    idxc = jnp.stack([gidx[:1], seg_trash[None]], axis=4)
    ones_rows = jnp.ones((ROWLEN, L), jnp.float32)
    zeros_f32 = jnp.zeros((ZCH, L), jnp.float32)
    zeros_s16 = jnp.zeros((ZCH, LB), jnp.int16)

    sc_agg = _make_sc_agg()
    cntp = _make_sc_cnt()(idxc, ones_rows, zeros_f32)
    cnt0 = jnp.repeat(cntp[0, :, 0].reshape(N, R), LB, axis=1)
    cnt1 = jnp.repeat(cntp[1, :, 0].reshape(N, R), LB, axis=1)

    maxx, maxcnt = _prep_stats(x, cntp[0].reshape(N, F), cntp[1].reshape(N, F))
    maxcnt = jnp.maximum(maxcnt[0, 0], 1.0)

    def quant(h, mabs):
        s = 32000.0 / (maxcnt * jnp.maximum(mabs, 1e-30))
        hq = jnp.round(h * s).astype(jnp.int16).reshape(N * NSLICE, LB)
        hq = jnp.concatenate([hq, jnp.zeros((8, LB), jnp.int16)])
        return hq, (1.0 / s).reshape(1, 1)

    xq, inv_s0 = quant(x, maxx[0, 0])
    a0 = sc_agg(xq, idx2, zeros_s16)
    h0, m0 = _tc_layer(a0, cnt0, cnt1, inv_s0, x, basis0, comp0, root0,
                       bias0, 'relu', True)
    h0q, inv_s1 = quant(h0, m0[0, 0])
    a1 = sc_agg(h0q, idx2, zeros_s16)
    h1, m1 = _tc_layer(a1, cnt0, cnt1, inv_s1, h0, basis1, comp1, root1,
                       bias1, 'relu', True)
    h1q, inv_s2 = quant(h1, m1[0, 0])
    a2 = sc_agg(h1q, idx2, zeros_s16)
    (out,) = _tc_layer(a2, cnt0, cnt1, inv_s2, h1, basis2, comp2, root2,
                       bias2, 'tanh', False)
    return out


# speedup vs baseline: 1.0618x; 1.0618x over previous
import functools

import jax
import jax.numpy as jnp
from jax import lax
from jax.experimental import pallas as pl
from jax.experimental.pallas import tpu as pltpu
from jax.experimental.pallas import tpu_sc as plsc

N = 10000
E = 320000
R = 8
F = 128
L = 16
LB = 32
NSLICE = F // LB
NSEG = N * R
NC = 2
NS = 16
ROWLEN = 128
NBUF = 8
KROWS = 160
NBLK = KROWS // NBUF
E_PAD = NS * ROWLEN * KROWS
ACC_ROWS = NSEG + L
ZROWS_TILE = ACC_ROWS // NS
ZCH = ZROWS_TILE // 3
SEG_TILE = NSEG // NS
CBLK = NBLK // NC


def _make_sc_agg(interpret=False):
    mesh = plsc.VectorSubcoreMesh(core_axis_name="c", subcore_axis_name="s",
                                  num_cores=NC, num_subcores=NS)

    @functools.partial(
        pl.kernel,
        out_type=jax.ShapeDtypeStruct((NC, NSLICE, NSEG, LB), jnp.int16),
        mesh=mesh,
        interpret=interpret,
        compiler_params=pltpu.CompilerParams(use_tc_tiling_on_sc=False),
        scratch_types=[
            pltpu.VMEM_SHARED((ACC_ROWS, LB), jnp.int16),
            pltpu.VMEM((2, NBUF, 2, ROWLEN), jnp.int32),
            pltpu.VMEM((2, NBUF, ROWLEN, LB), jnp.int16),
            pltpu.SemaphoreType.DMA,
            pltpu.SemaphoreType.DMA,
            pltpu.SemaphoreType.DMA,
        ],
    )
    def sc_kernel(xv_hbm, idx2_hbm, zeros_hbm, a_out,
                  acc_sh, idx_v, row_v, gsem, ssem, isem):
        c = lax.axis_index("c")
        s = lax.axis_index("s")

        def zero_my_stripe():
            for kz in range(3):
                pltpu.sync_copy(
                    zeros_hbm, acc_sh.at[pl.ds(s * ZROWS_TILE + kz * ZCH, ZCH)])

        def iwait():
            pltpu.make_async_copy(idx2_hbm.at[0, 0, 0], idx_v.at[0],
                                  isem).wait()

        def gissue(h):
            for b in range(NBUF):
                pltpu.async_copy(xv_hbm.at[idx_v.at[h, b, 0]],
                                 row_v.at[h, b], gsem)

        def gdrain():
            for _ in range(NBUF):
                pltpu.make_async_copy(xv_hbm.at[pl.ds(0, ROWLEN)],
                                      row_v.at[0, 0], gsem).wait()

        def sissue(h):
            for b in range(NBUF):
                pltpu.async_copy(row_v.at[h, b],
                                 acc_sh.at[idx_v.at[h, b, 1]], ssem, add=True)

        def sdrain():
            for _ in range(NBUF):
                pltpu.make_async_copy(row_v.at[0, 0],
                                      acc_sh.at[pl.ds(0, ROWLEN)], ssem).wait()

        nhb = NBLK // NC
        blk_base = c * nhb

        for i in range(NSLICE):

            def istart(blk, h):
                pltpu.async_copy(idx2_hbm.at[i, s, blk_base + blk],
                                 idx_v.at[h], isem)

            zero_my_stripe()
            plsc.subcore_barrier()

            istart(0, 0)
            iwait()
            gissue(0)
            istart(1, 1)

            def pairbody(ip, carry):
                for h in range(2):
                    jg = ip * 2 + h
                    gdrain()
                    sissue(h)

                    @pl.when(jg + 1 < nhb)
                    def _():
                        iwait()

                    @pl.when(jg + 1 < nhb)
                    def _():
                        gissue(1 - h)

                    sdrain()

                    @pl.when(jg + 2 < nhb)
                    def _():
                        istart(jg + 2, h)
                return carry

            lax.fori_loop(0, nhb // 2, pairbody, 0)
            plsc.subcore_barrier()
            pltpu.sync_copy(acc_sh.at[pl.ds(s * SEG_TILE, SEG_TILE)],
                            a_out.at[c, i, pl.ds(s * SEG_TILE, SEG_TILE)])
            plsc.subcore_barrier()

    return sc_kernel


def _make_sc_cnt(interpret=False):
    mesh = plsc.VectorSubcoreMesh(core_axis_name="c", subcore_axis_name="s",
                                  num_cores=NC, num_subcores=NS)

    @functools.partial(
        pl.kernel,
        out_type=jax.ShapeDtypeStruct((NC, NSEG, L), jnp.float32),
        mesh=mesh,
        interpret=interpret,
        compiler_params=pltpu.CompilerParams(use_tc_tiling_on_sc=False),
        scratch_types=[
            pltpu.VMEM_SHARED((ACC_ROWS, L), jnp.float32),
            pltpu.VMEM((2, NBUF, 2, ROWLEN), jnp.int32),
            pltpu.VMEM((ROWLEN, L), jnp.float32),
            pltpu.SemaphoreType.DMA,
            pltpu.SemaphoreType.DMA,
        ],
    )
    def cnt_kernel(idx2_hbm, ones_hbm, zeros_hbm, cnt_out,
                   acc_sh, idx_v, ones_v, ssem, isem):
        c = lax.axis_index("c")
        s = lax.axis_index("s")
        blk0 = c * CBLK

        pltpu.sync_copy(ones_hbm, ones_v)
        for kz in range(3):
            pltpu.sync_copy(
                zeros_hbm, acc_sh.at[pl.ds(s * ZROWS_TILE + kz * ZCH, ZCH)])
        plsc.subcore_barrier()

        def iwait():
            pltpu.make_async_copy(idx2_hbm.at[0, 0, 0], idx_v.at[0],
                                  isem).wait()

        def sdrain():
            for _ in range(NBUF):
                pltpu.make_async_copy(ones_v, acc_sh.at[pl.ds(0, ROWLEN)],
                                      ssem).wait()

        pltpu.async_copy(idx2_hbm.at[0, s, blk0], idx_v.at[0], isem)

        def cpair(ip, carry):
            for h in range(2):
                jg = ip * 2 + h
                iwait()

                @pl.when(jg + 1 < CBLK)
                def _():
                    pltpu.async_copy(idx2_hbm.at[0, s, blk0 + jg + 1],
                                     idx_v.at[1 - h], isem)

                for b in range(NBUF):
                    pltpu.async_copy(ones_v, acc_sh.at[idx_v.at[h, b, 1]],
                                     ssem, add=True)
                sdrain()
            return carry

        lax.fori_loop(0, CBLK // 2, cpair, 0)
        plsc.subcore_barrier()
        pltpu.sync_copy(acc_sh.at[pl.ds(s * SEG_TILE, SEG_TILE)],
                        cnt_out.at[c, pl.ds(s * SEG_TILE, SEG_TILE)])

    return cnt_kernel


def _prep_stats(xin, c0v, c1v, interpret=False):
    def body(x_ref, c0_ref, c1_ref, mx_ref, mc_ref):
        mx_ref[...] = jnp.max(jnp.abs(x_ref[...])).reshape(1, 1)
        mc_ref[...] = jnp.maximum(jnp.max(c0_ref[...]),
                                  jnp.max(c1_ref[...])).reshape(1, 1)

    return pl.pallas_call(
        body,
        out_shape=[jax.ShapeDtypeStruct((1, 1), jnp.float32),
                   jax.ShapeDtypeStruct((1, 1), jnp.float32)],
        interpret=interpret,
    )(xin, c0v, c1v)


def _tc_layer(a, cnt0, cnt1, inv_s, xin, basis, comp, root, bias, act,
              want_max, interpret=False):
    out_dim = root.shape[1]
    w = jnp.einsum('rb,bio->rio', comp, basis)
    w2 = w.reshape(R, NSLICE, LB, out_dim).transpose(1, 0, 2, 3).reshape(
        NSLICE, R * LB, out_dim)
    a3 = a.reshape(NC, NSLICE, N, R * LB)
    bias2 = bias.reshape(1, out_dim)
    nb = 1000
    grid = (N // nb,)

    def body(a_ref, c0_ref, c1_ref, is_ref, x_ref, w2_ref, root_ref, b_ref,
             *o_refs):
        recip = is_ref[...] / jnp.maximum(c0_ref[...] + c1_ref[...], 1.0)
        acc = jnp.dot(x_ref[...], root_ref[...],
                      preferred_element_type=jnp.float32)
        for p in range(NSLICE):
            ap = (a_ref[0, p].astype(jnp.float32)
                  + a_ref[1, p].astype(jnp.float32)) * recip
            acc += jnp.dot(ap, w2_ref[p], preferred_element_type=jnp.float32)
        acc += b_ref[...]
        if act == 'relu':
            acc = jnp.maximum(acc, 0.0)
        else:
            acc = jnp.tanh(acc)
        o_refs[0][...] = acc
        if want_max:
            i = pl.program_id(0)
            bmax = jnp.max(acc).reshape(1, 1)

            @pl.when(i == 0)
            def _():
                o_refs[1][...] = bmax

            @pl.when(i > 0)
            def _():
                o_refs[1][...] = jnp.maximum(o_refs[1][...], bmax)

    out_shapes = [jax.ShapeDtypeStruct((N, out_dim), jnp.float32)]
    out_specs = [pl.BlockSpec((nb, out_dim), lambda i: (i, 0))]
    if want_max:
        out_shapes.append(jax.ShapeDtypeStruct((1, 1), jnp.float32))
        out_specs.append(pl.BlockSpec((1, 1), lambda i: (0, 0)))

    return pl.pallas_call(
        body,
        grid=grid,
        in_specs=[
            pl.BlockSpec((NC, NSLICE, nb, R * LB), lambda i: (0, 0, i, 0)),
            pl.BlockSpec((nb, R * LB), lambda i: (i, 0)),
            pl.BlockSpec((nb, R * LB), lambda i: (i, 0)),
            pl.BlockSpec((1, 1), lambda i: (0, 0)),
            pl.BlockSpec((nb, F), lambda i: (i, 0)),
            pl.BlockSpec((NSLICE, R * LB, out_dim), lambda i: (0, 0, 0)),
            pl.BlockSpec((F, out_dim), lambda i: (0, 0)),
            pl.BlockSpec((1, out_dim), lambda i: (0, 0)),
        ],
        out_specs=out_specs,
        out_shape=out_shapes,
        interpret=interpret,
    )(a3, cnt0, cnt1, inv_s, xin, w2, root, bias2)


def kernel(x, edge_index, edge_type,
           basis0, comp0, root0, bias0,
           basis1, comp1, root1, bias1,
           basis2, comp2, root2, bias2):
    src = edge_index[0].astype(jnp.int32)
    dst = edge_index[1].astype(jnp.int32)
    seg = dst * R + edge_type.astype(jnp.int32)

    pad = E_PAD - E
    seg_p = jnp.concatenate(
        [seg, jnp.full((pad,), NSEG, jnp.int32)]).reshape(
            NS, NBLK, NBUF, ROWLEN)
    src_p = jnp.concatenate([src, jnp.zeros((pad,), jnp.int32)])
    src_rs = src_p.reshape(NS, NBLK, NBUF, ROWLEN)
    gidx = (src_rs[None] * NSLICE
            + jnp.arange(NSLICE, dtype=jnp.int32)[:, None, None, None, None])
    idx2 = jnp.stack(
        [gidx, jnp.broadcast_to(seg_p[None], gidx.shape)], axis=4)
    ones_rows = jnp.ones((ROWLEN, L), jnp.float32)
    zeros_f32 = jnp.zeros((ZCH, L), jnp.float32)
    zeros_s16 = jnp.zeros((ZCH, LB), jnp.int16)

    sc_agg = _make_sc_agg()
    cntp = _make_sc_cnt()(idx2, ones_rows, zeros_f32)
    cnt0 = jnp.repeat(cntp[0, :, 0].reshape(N, R), LB, axis=1)
    cnt1 = jnp.repeat(cntp[1, :, 0].reshape(N, R), LB, axis=1)

    maxx, maxcnt = _prep_stats(x, cntp[0].reshape(N, F), cntp[1].reshape(N, F))
    maxcnt = jnp.maximum(maxcnt[0, 0], 1.0)

    def quant(h, mabs):
        s = 32000.0 / (maxcnt * jnp.maximum(mabs, 1e-30))
        hq = jnp.round(h * s).astype(jnp.int16)
        return hq.reshape(N * NSLICE, LB), (1.0 / s).reshape(1, 1)

    xq, inv_s0 = quant(x, maxx[0, 0])
    a0 = sc_agg(xq, idx2, zeros_s16)
    h0, m0 = _tc_layer(a0, cnt0, cnt1, inv_s0, x, basis0, comp0, root0,
                       bias0, 'relu', True)
    h0q, inv_s1 = quant(h0, m0[0, 0])
    a1 = sc_agg(h0q, idx2, zeros_s16)
    h1, m1 = _tc_layer(a1, cnt0, cnt1, inv_s1, h0, basis1, comp1, root1,
                       bias1, 'relu', True)
    h1q, inv_s2 = quant(h1, m1[0, 0])
    a2 = sc_agg(h1q, idx2, zeros_s16)
    (out,) = _tc_layer(a2, cnt0, cnt1, inv_s2, h1, basis2, comp2, root2,
                       bias2, 'tanh', False)
    return out
